# baseline (speedup 1.0000x reference)
import jax
import jax.numpy as jnp
from jax.experimental import pallas as pl

_N, _D, _E = 2048, 768, 8


def _gate_kernel(x_ref, wg_ref, bg_ref, g4_ref, g1_ref, gloss_ref):
    x = x_ref[:]
    logits = jnp.dot(x, wg_ref[:], preferred_element_type=jnp.float32) + bg_ref[:]
    m = jnp.max(logits, axis=-1, keepdims=True)
    ex = jnp.exp(logits - m)
    gs = ex / jnp.sum(ex, axis=-1, keepdims=True)
    e_iota = jax.lax.broadcasted_iota(jnp.int32, gs.shape, 1)
    rank = jnp.zeros(gs.shape, dtype=jnp.int32)
    for j in range(_E):
        gj = gs[:, j:j + 1]
        hit = (gj > gs) | ((gj == gs) & (j < e_iota))
        rank = rank + hit.astype(jnp.int32)
    mask4 = (rank < 4).astype(jnp.float32)
    mask1 = (rank < 1).astype(jnp.float32)
    g4_ref[:] = gs * mask4
    g1_ref[:] = gs * mask1
    density = jnp.mean(mask4, axis=0, keepdims=True)
    proxy = jnp.mean(gs, axis=0, keepdims=True)
    gloss_ref[:] = jnp.reshape(jnp.mean(density * proxy) * float(_E * _E), (1, 1))


def _x_kernel(x_ref, wmu_ref, bmu_ref, wlv_ref, blv_ref, g4_ref,
              ox_ref, lacc_ref):
    e = pl.program_id(0)
    x = x_ref[:]
    xb = x.astype(jnp.bfloat16)
    a = jnp.dot(xb, wmu_ref[0], preferred_element_type=jnp.float32)
    b = jnp.dot(xb, wlv_ref[0], preferred_element_type=jnp.float32)
    mu = a + bmu_ref[0] + x
    lv = b + blv_ref[0]
    exl = jnp.exp(lv)
    elem = (mu * mu + exl - lv - 1.0) * 0.5
    sel = (jax.lax.broadcasted_iota(jnp.int32, (_N, _E), 1) == e).astype(jnp.float32)
    g4 = jnp.sum(g4_ref[:] * sel, axis=1, keepdims=True)
    kl_sum = jnp.sum(elem)
    u = jnp.sum(exl, axis=1, keepdims=True)
    u_sum = jnp.sum(g4 * u)
    contrib = kl_sum / float(_N * _E) + u_sum / float(_N)

    @pl.when(e == 0)
    def _():
        ox_ref[:] = g4 * mu
        lacc_ref[:] = jnp.reshape(contrib, (1, 1))

    @pl.when(e != 0)
    def _():
        ox_ref[:] += g4 * mu
        lacc_ref[:] += jnp.reshape(contrib, (1, 1))


def _yz_kernel(y_ref, z_ref, wmu_ref, bmu_ref, g1_ref, oy_ref, oz_ref):
    e = pl.program_id(0)
    wmu = wmu_ref[0]
    sel = (jax.lax.broadcasted_iota(jnp.int32, (_N, _E), 1) == e).astype(jnp.float32)
    g1 = jnp.sum(g1_ref[:] * sel, axis=1, keepdims=True)
    muy = (jnp.dot(y_ref[:].astype(jnp.bfloat16), wmu, preferred_element_type=jnp.float32)
           + bmu_ref[0] + y_ref[:])
    muz = (jnp.dot(z_ref[:].astype(jnp.bfloat16), wmu, preferred_element_type=jnp.float32)
           + bmu_ref[0] + z_ref[:])

    @pl.when(e == 0)
    def _():
        oy_ref[:] = g1 * muy
        oz_ref[:] = g1 * muz

    @pl.when(e != 0)
    def _():
        oy_ref[:] += g1 * muy
        oz_ref[:] += g1 * muz


def kernel(x, y, z, Wg, bg, Wmu, bmu, Wlv, blv):
    f32 = jnp.float32
    g4, g1, gloss = pl.pallas_call(
        _gate_kernel,
        out_shape=(
            jax.ShapeDtypeStruct((_N, _E), f32),
            jax.ShapeDtypeStruct((_N, _E), f32),
            jax.ShapeDtypeStruct((1, 1), f32),
        ),
    )(x, Wg, bg.reshape(1, _E))

    bmu3 = bmu.reshape(_E, 1, _D)
    blv3 = blv.reshape(_E, 1, _D)
    wmu_bf = Wmu.astype(jnp.bfloat16)
    wlv_bf = Wlv.astype(jnp.bfloat16)

    ox, lacc = pl.pallas_call(
        _x_kernel,
        grid=(_E,),
        in_specs=[
            pl.BlockSpec((_N, _D), lambda e: (0, 0)),
            pl.BlockSpec((1, _D, _D), lambda e: (e, 0, 0)),
            pl.BlockSpec((1, 1, _D), lambda e: (e, 0, 0)),
            pl.BlockSpec((1, _D, _D), lambda e: (e, 0, 0)),
            pl.BlockSpec((1, 1, _D), lambda e: (e, 0, 0)),
            pl.BlockSpec((_N, _E), lambda e: (0, 0)),
        ],
        out_specs=(
            pl.BlockSpec((_N, _D), lambda e: (0, 0)),
            pl.BlockSpec((1, 1), lambda e: (0, 0)),
        ),
        out_shape=(
            jax.ShapeDtypeStruct((_N, _D), f32),
            jax.ShapeDtypeStruct((1, 1), f32),
        ),
    )(x, wmu_bf, bmu3, wlv_bf, blv3, g4)

    oy, oz = pl.pallas_call(
        _yz_kernel,
        grid=(_E,),
        in_specs=[
            pl.BlockSpec((_N, _D), lambda e: (0, 0)),
            pl.BlockSpec((_N, _D), lambda e: (0, 0)),
            pl.BlockSpec((1, _D, _D), lambda e: (e, 0, 0)),
            pl.BlockSpec((1, 1, _D), lambda e: (e, 0, 0)),
            pl.BlockSpec((_N, _E), lambda e: (0, 0)),
        ],
        out_specs=(
            pl.BlockSpec((_N, _D), lambda e: (0, 0)),
            pl.BlockSpec((_N, _D), lambda e: (0, 0)),
        ),
        out_shape=(
            jax.ShapeDtypeStruct((_N, _D), f32),
            jax.ShapeDtypeStruct((_N, _D), f32),
        ),
    )(y, z, wmu_bf, bmu3, g1)

    loss = gloss[0, 0] + lacc[0, 0]
    return ox, oy, oz, loss


# speedup vs baseline: 1.1018x; 1.1018x over previous
import jax
import jax.numpy as jnp
from jax.experimental import pallas as pl

_N, _D, _E = 2048, 768, 8


def _gate_kernel(x_ref, wg_ref, bg_ref, g4_ref, g1_ref, gloss_ref):
    x = x_ref[:]
    logits = jnp.dot(x, wg_ref[:], preferred_element_type=jnp.float32) + bg_ref[:]
    m = jnp.max(logits, axis=-1, keepdims=True)
    ex = jnp.exp(logits - m)
    gs = ex / jnp.sum(ex, axis=-1, keepdims=True)
    e_iota = jax.lax.broadcasted_iota(jnp.int32, gs.shape, 1)
    rank = jnp.zeros(gs.shape, dtype=jnp.int32)
    for j in range(_E):
        gj = gs[:, j:j + 1]
        hit = (gj > gs) | ((gj == gs) & (j < e_iota))
        rank = rank + hit.astype(jnp.int32)
    mask4 = (rank < 4).astype(jnp.float32)
    mask1 = (rank < 1).astype(jnp.float32)
    g4_ref[:] = gs * mask4
    g1_ref[:] = gs * mask1
    density = jnp.mean(mask4, axis=0, keepdims=True)
    proxy = jnp.mean(gs, axis=0, keepdims=True)
    gloss_ref[:] = jnp.reshape(jnp.mean(density * proxy) * float(_E * _E), (1, 1))


def _x_kernel(x_ref, wmu_ref, bmu_ref, wlv_ref, blv_ref, g4_ref,
              ox_ref, lacc_ref):
    e = pl.program_id(0)
    x = x_ref[:]
    a = jnp.dot(x, wmu_ref[0], preferred_element_type=jnp.float32)
    b = jnp.dot(x, wlv_ref[0], preferred_element_type=jnp.float32)
    mu = a + bmu_ref[0] + x
    lv = b + blv_ref[0]
    exl = jnp.exp(lv)
    elem = (mu * mu + exl - lv - 1.0) * 0.5
    sel = (jax.lax.broadcasted_iota(jnp.int32, (_N, _E), 1) == e).astype(jnp.float32)
    g4 = jnp.sum(g4_ref[:] * sel, axis=1, keepdims=True)
    kl_sum = jnp.sum(elem)
    u = jnp.sum(exl, axis=1, keepdims=True)
    u_sum = jnp.sum(g4 * u)
    contrib = kl_sum / float(_N * _E) + u_sum / float(_N)

    @pl.when(e == 0)
    def _():
        ox_ref[:] = g4 * mu
        lacc_ref[:] = jnp.reshape(contrib, (1, 1))

    @pl.when(e != 0)
    def _():
        ox_ref[:] += g4 * mu
        lacc_ref[:] += jnp.reshape(contrib, (1, 1))


def _yz_kernel(y_ref, z_ref, wmu_ref, bmu_ref, g1_ref, oy_ref, oz_ref):
    e = pl.program_id(0)
    wmu = wmu_ref[0]
    sel = (jax.lax.broadcasted_iota(jnp.int32, (_N, _E), 1) == e).astype(jnp.float32)
    g1 = jnp.sum(g1_ref[:] * sel, axis=1, keepdims=True)
    muy = jnp.dot(y_ref[:], wmu, preferred_element_type=jnp.float32) + bmu_ref[0] + y_ref[:]
    muz = jnp.dot(z_ref[:], wmu, preferred_element_type=jnp.float32) + bmu_ref[0] + z_ref[:]

    @pl.when(e == 0)
    def _():
        oy_ref[:] = g1 * muy
        oz_ref[:] = g1 * muz

    @pl.when(e != 0)
    def _():
        oy_ref[:] += g1 * muy
        oz_ref[:] += g1 * muz


def kernel(x, y, z, Wg, bg, Wmu, bmu, Wlv, blv):
    f32 = jnp.float32
    g4, g1, gloss = pl.pallas_call(
        _gate_kernel,
        out_shape=(
            jax.ShapeDtypeStruct((_N, _E), f32),
            jax.ShapeDtypeStruct((_N, _E), f32),
            jax.ShapeDtypeStruct((1, 1), f32),
        ),
    )(x, Wg, bg.reshape(1, _E))

    bmu3 = bmu.reshape(_E, 1, _D)
    blv3 = blv.reshape(_E, 1, _D)

    ox, lacc = pl.pallas_call(
        _x_kernel,
        grid=(_E,),
        in_specs=[
            pl.BlockSpec((_N, _D), lambda e: (0, 0)),
            pl.BlockSpec((1, _D, _D), lambda e: (e, 0, 0)),
            pl.BlockSpec((1, 1, _D), lambda e: (e, 0, 0)),
            pl.BlockSpec((1, _D, _D), lambda e: (e, 0, 0)),
            pl.BlockSpec((1, 1, _D), lambda e: (e, 0, 0)),
            pl.BlockSpec((_N, _E), lambda e: (0, 0)),
        ],
        out_specs=(
            pl.BlockSpec((_N, _D), lambda e: (0, 0)),
            pl.BlockSpec((1, 1), lambda e: (0, 0)),
        ),
        out_shape=(
            jax.ShapeDtypeStruct((_N, _D), f32),
            jax.ShapeDtypeStruct((1, 1), f32),
        ),
    )(x, Wmu, bmu3, Wlv, blv3, g4)

    oy, oz = pl.pallas_call(
        _yz_kernel,
        grid=(_E,),
        in_specs=[
            pl.BlockSpec((_N, _D), lambda e: (0, 0)),
            pl.BlockSpec((_N, _D), lambda e: (0, 0)),
            pl.BlockSpec((1, _D, _D), lambda e: (e, 0, 0)),
            pl.BlockSpec((1, 1, _D), lambda e: (e, 0, 0)),
            pl.BlockSpec((_N, _E), lambda e: (0, 0)),
        ],
        out_specs=(
            pl.BlockSpec((_N, _D), lambda e: (0, 0)),
            pl.BlockSpec((_N, _D), lambda e: (0, 0)),
        ),
        out_shape=(
            jax.ShapeDtypeStruct((_N, _D), f32),
            jax.ShapeDtypeStruct((_N, _D), f32),
        ),
    )(y, z, Wmu, bmu3, g1)

    loss = gloss[0, 0] + lacc[0, 0]
    return ox, oy, oz, loss


# speedup vs baseline: 1.1023x; 1.0005x over previous
import jax
import jax.numpy as jnp
from jax.experimental import pallas as pl

_N, _D, _E = 2048, 768, 8


def _gate_kernel(x_ref, wg_ref, bg_ref, g4_ref, g1_ref, gloss_ref):
    x = x_ref[:]
    logits = jnp.dot(x, wg_ref[:], preferred_element_type=jnp.float32) + bg_ref[:]
    m = jnp.max(logits, axis=-1, keepdims=True)
    ex = jnp.exp(logits - m)
    gs = ex / jnp.sum(ex, axis=-1, keepdims=True)
    e_iota = jax.lax.broadcasted_iota(jnp.int32, gs.shape, 1)
    rank = jnp.zeros(gs.shape, dtype=jnp.int32)
    for j in range(_E):
        gj = gs[:, j:j + 1]
        hit = (gj > gs) | ((gj == gs) & (j < e_iota))
        rank = rank + hit.astype(jnp.int32)
    mask4 = (rank < 4).astype(jnp.float32)
    mask1 = (rank < 1).astype(jnp.float32)
    g4_ref[:] = gs * mask4
    g1_ref[:] = gs * mask1
    density = jnp.mean(mask4, axis=0, keepdims=True)
    proxy = jnp.mean(gs, axis=0, keepdims=True)
    gloss_ref[:] = jnp.reshape(jnp.mean(density * proxy) * float(_E * _E), (1, 1))


def _x_kernel(x_ref, wmu_ref, bmu_ref, wlv_ref, blv_ref, g4_ref,
              ox_ref, lacc_ref):
    e = pl.program_id(0)
    x = x_ref[:]
    xb = x.astype(jnp.bfloat16)
    a = jnp.dot(xb, wmu_ref[0].astype(jnp.bfloat16), preferred_element_type=jnp.float32)
    b = jnp.dot(xb, wlv_ref[0].astype(jnp.bfloat16), preferred_element_type=jnp.float32)
    mu = a + bmu_ref[0] + x
    lv = b + blv_ref[0]
    exl = jnp.exp(lv)
    elem = (mu * mu + exl - lv - 1.0) * 0.5
    sel = (jax.lax.broadcasted_iota(jnp.int32, (_N, _E), 1) == e).astype(jnp.float32)
    g4 = jnp.sum(g4_ref[:] * sel, axis=1, keepdims=True)
    kl_sum = jnp.sum(elem)
    u = jnp.sum(exl, axis=1, keepdims=True)
    u_sum = jnp.sum(g4 * u)
    contrib = kl_sum / float(_N * _E) + u_sum / float(_N)

    @pl.when(e == 0)
    def _():
        ox_ref[:] = g4 * mu
        lacc_ref[:] = jnp.reshape(contrib, (1, 1))

    @pl.when(e != 0)
    def _():
        ox_ref[:] += g4 * mu
        lacc_ref[:] += jnp.reshape(contrib, (1, 1))


def _yz_kernel(y_ref, z_ref, wmu_ref, bmu_ref, g1_ref, oy_ref, oz_ref):
    e = pl.program_id(0)
    wmu = wmu_ref[0]
    sel = (jax.lax.broadcasted_iota(jnp.int32, (_N, _E), 1) == e).astype(jnp.float32)
    g1 = jnp.sum(g1_ref[:] * sel, axis=1, keepdims=True)
    wmb = wmu.astype(jnp.bfloat16)
    muy = (jnp.dot(y_ref[:].astype(jnp.bfloat16), wmb, preferred_element_type=jnp.float32)
           + bmu_ref[0] + y_ref[:])
    muz = (jnp.dot(z_ref[:].astype(jnp.bfloat16), wmb, preferred_element_type=jnp.float32)
           + bmu_ref[0] + z_ref[:])

    @pl.when(e == 0)
    def _():
        oy_ref[:] = g1 * muy
        oz_ref[:] = g1 * muz

    @pl.when(e != 0)
    def _():
        oy_ref[:] += g1 * muy
        oz_ref[:] += g1 * muz


def kernel(x, y, z, Wg, bg, Wmu, bmu, Wlv, blv):
    f32 = jnp.float32
    g4, g1, gloss = pl.pallas_call(
        _gate_kernel,
        out_shape=(
            jax.ShapeDtypeStruct((_N, _E), f32),
            jax.ShapeDtypeStruct((_N, _E), f32),
            jax.ShapeDtypeStruct((1, 1), f32),
        ),
    )(x, Wg, bg.reshape(1, _E))

    bmu3 = bmu.reshape(_E, 1, _D)
    blv3 = blv.reshape(_E, 1, _D)

    ox, lacc = pl.pallas_call(
        _x_kernel,
        grid=(_E,),
        in_specs=[
            pl.BlockSpec((_N, _D), lambda e: (0, 0)),
            pl.BlockSpec((1, _D, _D), lambda e: (e, 0, 0)),
            pl.BlockSpec((1, 1, _D), lambda e: (e, 0, 0)),
            pl.BlockSpec((1, _D, _D), lambda e: (e, 0, 0)),
            pl.BlockSpec((1, 1, _D), lambda e: (e, 0, 0)),
            pl.BlockSpec((_N, _E), lambda e: (0, 0)),
        ],
        out_specs=(
            pl.BlockSpec((_N, _D), lambda e: (0, 0)),
            pl.BlockSpec((1, 1), lambda e: (0, 0)),
        ),
        out_shape=(
            jax.ShapeDtypeStruct((_N, _D), f32),
            jax.ShapeDtypeStruct((1, 1), f32),
        ),
    )(x, Wmu, bmu3, Wlv, blv3, g4)

    oy, oz = pl.pallas_call(
        _yz_kernel,
        grid=(_E,),
        in_specs=[
            pl.BlockSpec((_N, _D), lambda e: (0, 0)),
            pl.BlockSpec((_N, _D), lambda e: (0, 0)),
            pl.BlockSpec((1, _D, _D), lambda e: (e, 0, 0)),
            pl.BlockSpec((1, 1, _D), lambda e: (e, 0, 0)),
            pl.BlockSpec((_N, _E), lambda e: (0, 0)),
        ],
        out_specs=(
            pl.BlockSpec((_N, _D), lambda e: (0, 0)),
            pl.BlockSpec((_N, _D), lambda e: (0, 0)),
        ),
        out_shape=(
            jax.ShapeDtypeStruct((_N, _D), f32),
            jax.ShapeDtypeStruct((_N, _D), f32),
        ),
    )(y, z, Wmu, bmu3, g1)

    loss = gloss[0, 0] + lacc[0, 0]
    return ox, oy, oz, loss


# speedup vs baseline: 1.3304x; 1.2069x over previous
import jax
import jax.numpy as jnp
from jax.experimental import pallas as pl

_N, _D, _E = 2048, 768, 8
_BN = 256


def _x_kernel(x_ref, wg_ref, bg_ref, wmu_ref, bmu_ref, wlv_ref, blv_ref,
              ox_ref, g4_ref, g1_ref, lacc_ref):
    e = pl.program_id(0)
    x = x_ref[:]

    @pl.when(e == 0)
    def _():
        logits = jnp.dot(x, wg_ref[:], preferred_element_type=jnp.float32) + bg_ref[:]
        mx = jnp.max(logits, axis=-1, keepdims=True)
        exg = jnp.exp(logits - mx)
        gs = exg / jnp.sum(exg, axis=-1, keepdims=True)
        e_iota = jax.lax.broadcasted_iota(jnp.int32, gs.shape, 1)
        rank = jnp.zeros(gs.shape, dtype=jnp.int32)
        for j in range(_E):
            gj = gs[:, j:j + 1]
            hit = (gj > gs) | ((gj == gs) & (j < e_iota))
            rank = rank + hit.astype(jnp.int32)
        mask4 = (rank < 4).astype(jnp.float32)
        g4_ref[:] = gs * mask4
        g1_ref[:] = gs * (rank < 1).astype(jnp.float32)
        density = jnp.mean(mask4, axis=0, keepdims=True)
        proxy = jnp.mean(gs, axis=0, keepdims=True)
        lacc_ref[:] = jnp.reshape(jnp.mean(density * proxy) * float(_E * _E), (1, 1))

    a = jnp.dot(x, wmu_ref[0], preferred_element_type=jnp.float32)
    b = jnp.dot(x, wlv_ref[0], preferred_element_type=jnp.float32)
    mu = a + bmu_ref[0] + x
    exl = jnp.exp(b + blv_ref[0])
    sel = (jax.lax.broadcasted_iota(jnp.int32, (_N, _E), 1) == e).astype(jnp.float32)
    g4 = jnp.sum(g4_ref[:] * sel, axis=1, keepdims=True)
    s_mu2 = jnp.sum(mu * mu, axis=1, keepdims=True)
    u = jnp.sum(exl, axis=1, keepdims=True)
    s_b = jnp.sum(b, axis=1, keepdims=True)
    kl_sum = 0.5 * (jnp.sum(s_mu2) + jnp.sum(u) - jnp.sum(s_b)
                    - float(_N) * jnp.sum(blv_ref[0]) - float(_N * _D))
    contrib = kl_sum / float(_N * _E) + jnp.sum(g4 * u) / float(_N)

    @pl.when(e == 0)
    def _():
        ox_ref[:] = g4 * mu
        lacc_ref[:] += jnp.reshape(contrib, (1, 1))

    @pl.when(e != 0)
    def _():
        ox_ref[:] += g4 * mu
        lacc_ref[:] += jnp.reshape(contrib, (1, 1))


def _yz_kernel(y_ref, z_ref, wms_ref, bmu_ref, g1_ref, oy_ref, oz_ref):
    g1 = g1_ref[:]
    gbias = jnp.dot(g1, bmu_ref[:], preferred_element_type=jnp.float32)
    g1sum = jnp.sum(g1, axis=1, keepdims=True)
    wms = wms_ref[:]
    for src, dst in ((y_ref, oy_ref), (z_ref, oz_ref)):
        v = src[:]
        parts = [g1[:, e:e + 1] * v for e in range(_E)]
        vp = jnp.concatenate(parts, axis=1)
        dst[:] = jnp.dot(vp, wms, preferred_element_type=jnp.float32) + gbias + g1sum * v


def kernel(x, y, z, Wg, bg, Wmu, bmu, Wlv, blv):
    f32 = jnp.float32
    bmu3 = bmu.reshape(_E, 1, _D)
    blv3 = blv.reshape(_E, 1, _D)

    ox, g4, g1, lacc = pl.pallas_call(
        _x_kernel,
        grid=(_E,),
        in_specs=[
            pl.BlockSpec((_N, _D), lambda e: (0, 0)),
            pl.BlockSpec((_D, _E), lambda e: (0, 0)),
            pl.BlockSpec((1, _E), lambda e: (0, 0)),
            pl.BlockSpec((1, _D, _D), lambda e: (e, 0, 0)),
            pl.BlockSpec((1, 1, _D), lambda e: (e, 0, 0)),
            pl.BlockSpec((1, _D, _D), lambda e: (e, 0, 0)),
            pl.BlockSpec((1, 1, _D), lambda e: (e, 0, 0)),
        ],
        out_specs=(
            pl.BlockSpec((_N, _D), lambda e: (0, 0)),
            pl.BlockSpec((_N, _E), lambda e: (0, 0)),
            pl.BlockSpec((_N, _E), lambda e: (0, 0)),
            pl.BlockSpec((1, 1), lambda e: (0, 0)),
        ),
        out_shape=(
            jax.ShapeDtypeStruct((_N, _D), f32),
            jax.ShapeDtypeStruct((_N, _E), f32),
            jax.ShapeDtypeStruct((_N, _E), f32),
            jax.ShapeDtypeStruct((1, 1), f32),
        ),
    )(x, Wg, bg.reshape(1, _E), Wmu, bmu3, Wlv, blv3)

    nt = _N // _BN
    oy, oz = pl.pallas_call(
        _yz_kernel,
        grid=(nt,),
        in_specs=[
            pl.BlockSpec((_BN, _D), lambda t: (t, 0)),
            pl.BlockSpec((_BN, _D), lambda t: (t, 0)),
            pl.BlockSpec((_E * _D, _D), lambda t: (0, 0)),
            pl.BlockSpec((_E, _D), lambda t: (0, 0)),
            pl.BlockSpec((_BN, _E), lambda t: (t, 0)),
        ],
        out_specs=(
            pl.BlockSpec((_BN, _D), lambda t: (t, 0)),
            pl.BlockSpec((_BN, _D), lambda t: (t, 0)),
        ),
        out_shape=(
            jax.ShapeDtypeStruct((_N, _D), f32),
            jax.ShapeDtypeStruct((_N, _D), f32),
        ),
    )(y, z, Wmu.reshape(_E * _D, _D), bmu, g1)

    loss = lacc[0, 0]
    return ox, oy, oz, loss


# speedup vs baseline: 1.5100x; 1.1350x over previous
import jax
import jax.numpy as jnp
from jax.experimental import pallas as pl

_N, _D, _E = 2048, 768, 8
_BN = 256
_NT = _N // _BN


def _fused_kernel(x_ref, y_ref, z_ref, wg_ref, bg_ref, wmu_ref, bmu_ref,
                  wlv_ref, blv_ref,
                  ox_ref, oy_ref, oz_ref, lacc_ref, macc_ref, pacc_ref):
    t = pl.program_id(0)
    f32 = jnp.float32
    x = x_ref[:]
    y = y_ref[:]
    z = z_ref[:]

    logits = jnp.dot(x, wg_ref[:], preferred_element_type=f32) + bg_ref[:]
    mx = jnp.max(logits, axis=-1, keepdims=True)
    exg = jnp.exp(logits - mx)
    gs = exg / jnp.sum(exg, axis=-1, keepdims=True)
    e_iota = jax.lax.broadcasted_iota(jnp.int32, gs.shape, 1)
    rank = jnp.zeros(gs.shape, dtype=jnp.int32)
    for j in range(_E):
        gj = gs[:, j:j + 1]
        hit = (gj > gs) | ((gj == gs) & (j < e_iota))
        rank = rank + hit.astype(jnp.int32)
    mask4 = (rank < 4).astype(f32)
    g4 = gs * mask4
    g1 = gs * (rank < 1).astype(f32)

    bmu = bmu_ref[:]
    g4sum = jnp.sum(g4, axis=1, keepdims=True)
    g1sum = jnp.sum(g1, axis=1, keepdims=True)
    acc_x = jnp.dot(g4, bmu, preferred_element_type=f32) + g4sum * x
    acc_y = jnp.dot(g1, bmu, preferred_element_type=f32) + g1sum * y
    acc_z = jnp.dot(g1, bmu, preferred_element_type=f32) + g1sum * z

    kl_part = jnp.float32(0.0)
    u_part = jnp.float32(0.0)
    for e in range(_E):
        w_e = wmu_ref[e]
        a = jnp.dot(x, w_e, preferred_element_type=f32)
        b = jnp.dot(x, wlv_ref[e], preferred_element_type=f32)
        mu = a + bmu[e:e + 1, :] + x
        exl = jnp.exp(b + blv_ref[e:e + 1, :])
        u_e = jnp.sum(exl, axis=1, keepdims=True)
        kl_part += (jnp.sum(jnp.sum(mu * mu, axis=1, keepdims=True))
                    + jnp.sum(u_e) - jnp.sum(jnp.sum(b, axis=1, keepdims=True))
                    - float(_BN) * jnp.sum(blv_ref[e:e + 1, :]) - float(_BN * _D))
        u_part += jnp.sum(g4[:, e:e + 1] * u_e)
        acc_x += g4[:, e:e + 1] * a
        acc_y += g1[:, e:e + 1] * jnp.dot(y, w_e, preferred_element_type=f32)
        acc_z += g1[:, e:e + 1] * jnp.dot(z, w_e, preferred_element_type=f32)

    ox_ref[:] = acc_x
    oy_ref[:] = acc_y
    oz_ref[:] = acc_z

    contrib = 0.5 * kl_part / float(_N * _E) + u_part / float(_N)
    msum = jnp.sum(mask4, axis=0, keepdims=True)
    psum = jnp.sum(gs, axis=0, keepdims=True)

    @pl.when(t == 0)
    def _():
        lacc_ref[:] = jnp.reshape(contrib, (1, 1))
        macc_ref[:] = msum
        pacc_ref[:] = psum

    @pl.when(t != 0)
    def _():
        lacc_ref[:] += jnp.reshape(contrib, (1, 1))
        macc_ref[:] += msum
        pacc_ref[:] += psum

    @pl.when(t == _NT - 1)
    def _():
        density = macc_ref[:] / float(_N)
        proxy = pacc_ref[:] / float(_N)
        gloss = jnp.mean(density * proxy) * float(_E * _E)
        lacc_ref[:] += jnp.reshape(gloss, (1, 1))


def kernel(x, y, z, Wg, bg, Wmu, bmu, Wlv, blv):
    f32 = jnp.float32
    ox, oy, oz, lacc, _, _ = pl.pallas_call(
        _fused_kernel,
        grid=(_NT,),
        in_specs=[
            pl.BlockSpec((_BN, _D), lambda t: (t, 0)),
            pl.BlockSpec((_BN, _D), lambda t: (t, 0)),
            pl.BlockSpec((_BN, _D), lambda t: (t, 0)),
            pl.BlockSpec((_D, _E), lambda t: (0, 0)),
            pl.BlockSpec((1, _E), lambda t: (0, 0)),
            pl.BlockSpec((_E, _D, _D), lambda t: (0, 0, 0)),
            pl.BlockSpec((_E, _D), lambda t: (0, 0)),
            pl.BlockSpec((_E, _D, _D), lambda t: (0, 0, 0)),
            pl.BlockSpec((_E, _D), lambda t: (0, 0)),
        ],
        out_specs=(
            pl.BlockSpec((_BN, _D), lambda t: (t, 0)),
            pl.BlockSpec((_BN, _D), lambda t: (t, 0)),
            pl.BlockSpec((_BN, _D), lambda t: (t, 0)),
            pl.BlockSpec((1, 1), lambda t: (0, 0)),
            pl.BlockSpec((1, _E), lambda t: (0, 0)),
            pl.BlockSpec((1, _E), lambda t: (0, 0)),
        ),
        out_shape=(
            jax.ShapeDtypeStruct((_N, _D), f32),
            jax.ShapeDtypeStruct((_N, _D), f32),
            jax.ShapeDtypeStruct((_N, _D), f32),
            jax.ShapeDtypeStruct((1, 1), f32),
            jax.ShapeDtypeStruct((1, _E), f32),
            jax.ShapeDtypeStruct((1, _E), f32),
        ),
    )(x, y, z, Wg, bg.reshape(1, _E), Wmu, bmu, Wlv, blv)

    loss = lacc[0, 0]
    return ox, oy, oz, loss
